# baseline (speedup 1.0000x reference)
import jax
import jax.numpy as jnp
from jax.experimental import pallas as pl
from jax.experimental.pallas import tpu as pltpu

_WIDTH = 1000
_FEATURE_DIM = 100000
_MAX_C0 = (_FEATURE_DIM // 128) * 128 - 128

_RB = 16


def _onehot_stripe(state_smem, state_vmem, out_ref):
    i = pl.program_id(0)
    out_ref[...] = jnp.zeros_like(out_ref)
    lanes = jax.lax.broadcasted_iota(jnp.int32, (8, 128), 1)

    for g in range(_RB // 8):
        row0 = i * _RB + g * 8
        sv = state_vmem[pl.ds(pl.multiple_of(row0, 8), 8), :]
        idxv = sv[:, 0:1] + _WIDTH * sv[:, 1:2]
        for r in range(8):
            c = state_smem[row0 + r, 0] + _WIDTH * state_smem[row0 + r, 1]
            c0 = jnp.minimum((c // 128) * 128, _MAX_C0)
            c0 = pl.multiple_of(c0, 128)
            patch = (lanes + c0 == idxv).astype(jnp.float32)
            out_ref[g * 8:(g + 1) * 8, pl.ds(c0, 128)] = patch


def kernel(state):
    n = state.shape[0]
    return pl.pallas_call(
        _onehot_stripe,
        grid=(n // _RB,),
        in_specs=[
            pl.BlockSpec(memory_space=pltpu.SMEM),
            pl.BlockSpec(memory_space=pltpu.VMEM),
        ],
        out_specs=pl.BlockSpec((_RB, _FEATURE_DIM), lambda i: (i, 0)),
        out_shape=jax.ShapeDtypeStruct((n, _FEATURE_DIM), jnp.float32),
    )(state, state)


# speedup vs baseline: 1.0058x; 1.0058x over previous
import jax
import jax.numpy as jnp
from jax.experimental import pallas as pl
from jax.experimental.pallas import tpu as pltpu

_WIDTH = 1000
_FEATURE_DIM = 100000
_MAX_C0 = (_FEATURE_DIM // 128) * 128 - 128

_RB = 8
_K = 8


def _onehot_ring(state_smem, state_vmem, out_hbm, scratch, sems):
    i = pl.program_id(0)
    nsteps = pl.num_programs(0)
    slot = jax.lax.rem(i, _K)

    def dma(k_slot, step):
        sbase = pl.multiple_of(k_slot * _RB, 8)
        return pltpu.make_async_copy(
            scratch.at[pl.ds(sbase, _RB), :],
            out_hbm.at[pl.ds(step * _RB, _RB), :],
            sems.at[k_slot],
        )

    @pl.when(i >= _K)
    def _():
        dma(slot, i - _K).wait()

    sbase = pl.multiple_of(slot * _RB, 8)
    scratch[pl.ds(sbase, _RB), :] = jnp.zeros((_RB, _FEATURE_DIM), jnp.float32)

    lanes = jax.lax.broadcasted_iota(jnp.int32, (8, 128), 1)
    row0 = i * _RB
    sv = state_vmem[pl.ds(pl.multiple_of(row0, 8), 8), :]
    idxv = sv[:, 0:1] + _WIDTH * sv[:, 1:2]
    for r in range(_RB):
        c = state_smem[row0 + r, 0] + _WIDTH * state_smem[row0 + r, 1]
        c0 = jnp.minimum((c // 128) * 128, _MAX_C0)
        c0 = pl.multiple_of(c0, 128)
        patch = (lanes + c0 == idxv).astype(jnp.float32)
        scratch[pl.ds(sbase, 8), pl.ds(c0, 128)] = patch

    dma(slot, i).start()

    @pl.when(i == nsteps - 1)
    def _():
        for k in range(_K):
            step = nsteps - _K + k
            dma(step % _K, step).wait()


def kernel(state):
    n = state.shape[0]
    return pl.pallas_call(
        _onehot_ring,
        grid=(n // _RB,),
        in_specs=[
            pl.BlockSpec(memory_space=pltpu.SMEM),
            pl.BlockSpec(memory_space=pltpu.VMEM),
        ],
        out_specs=pl.BlockSpec(memory_space=pl.ANY),
        out_shape=jax.ShapeDtypeStruct((n, _FEATURE_DIM), jnp.float32),
        scratch_shapes=[
            pltpu.VMEM((_K * _RB, _FEATURE_DIM), jnp.float32),
            pltpu.SemaphoreType.DMA((_K,)),
        ],
        compiler_params=pltpu.CompilerParams(
            dimension_semantics=("arbitrary",),
            vmem_limit_bytes=50 * 1024 * 1024,
        ),
    )(state, state)
